# baseline (speedup 1.0000x reference)
import functools
import math

import jax
import jax.numpy as jnp
from jax import lax
from jax.experimental import pallas as pl
from jax.experimental.pallas import tpu as pltpu
from jax.experimental.pallas import tpu_sc as plsc

B, S, D = 2, 2048, 1024
H, KV, DH = 16, 8, 64
E, TOPK, F = 8, 2, 2048
THETA, EPS = 10000.0, 1e-6
T = B * S
A = T * TOPK
BLK = 256
NB = (A // BLK) + E - 1
PAD = NB * BLK
RB = 512
NRB = T // RB

HALF = DH // 2
_LOG_THETA = math.log(THETA)


def _qkv_body(x_ref, w_ref, ln_ref, cos_ref, sin_ref, out_ref):
    x = x_ref[...]
    ms = jnp.mean(x * x, axis=-1, keepdims=True)
    xn = x * jax.lax.rsqrt(ms + EPS) * ln_ref[...]
    qkv = jnp.dot(xn.astype(jnp.bfloat16), w_ref[...].astype(jnp.bfloat16),
                  preferred_element_type=jnp.float32)

    QK = (H + KV) * DH
    NH = QK // DH
    qk = qkv[:, :QK]
    cosv = jnp.concatenate([cos_ref[...]] * NH, axis=1)
    sinv = jnp.concatenate([sin_ref[...]] * NH, axis=1)
    col = jax.lax.broadcasted_iota(jnp.int32, (RB, QK), 1)
    rolled_p = jnp.concatenate([qk[:, HALF:], qk[:, :HALF]], axis=1)
    rolled_m = jnp.concatenate([qk[:, -HALF:], qk[:, :-HALF]], axis=1)
    first_half = (col % DH) < HALF
    rh = jnp.where(first_half, -rolled_p, rolled_m)
    qk_rot = qk * cosv + rh * sinv
    out_ref[...] = jnp.concatenate([qk_rot, qkv[:, QK:]], axis=1)


def _qkv_call(x2d, wqkv, ln1_w, cos_t, sin_t):
    nsb = S // RB
    return pl.pallas_call(
        _qkv_body,
        grid=(NRB,),
        in_specs=[
            pl.BlockSpec((RB, D), lambda i: (i, 0)),
            pl.BlockSpec((D, (H + 2 * KV) * DH), lambda i: (0, 0)),
            pl.BlockSpec((1, D), lambda i: (0, 0)),
            pl.BlockSpec((RB, DH), lambda i: (i % nsb, 0)),
            pl.BlockSpec((RB, DH), lambda i: (i % nsb, 0)),
        ],
        out_specs=pl.BlockSpec((RB, (H + 2 * KV) * DH), lambda i: (i, 0)),
        out_shape=jax.ShapeDtypeStruct((T, (H + 2 * KV) * DH), jnp.float32),
    )(x2d, wqkv, ln1_w, cos_t, sin_t)


QBLK = 512
NQB = S // QBLK
KBLK = 512


def _attn_body(q_ref, k_ref, v_ref, o_ref):
    iq = pl.program_id(2)
    q = q_ref[0].astype(jnp.bfloat16)
    lane = jax.lax.broadcasted_iota(jnp.int32, (KBLK, 2 * DH), 1)
    is_v = lane < DH
    is_one = lane == DH

    def vaug(j):
        v = v_ref[0, pl.ds(j * KBLK, KBLK), :]
        vb = jnp.concatenate([v, v], axis=1)
        return jnp.where(is_v, vb, jnp.where(is_one, 1.0, 0.0)
                         ).astype(jnp.bfloat16)

    def scores(j):
        k = k_ref[0, pl.ds(j * KBLK, KBLK), :].astype(jnp.bfloat16)
        s = jax.lax.dot_general(q, k, (((1,), (1,)), ((), ())),
                                preferred_element_type=jnp.float32)
        return s * (1.0 / 8.0)

    def update(s, j, carry):
        m, acc = carry
        mj = jnp.max(s, axis=-1, keepdims=True)
        mn = jnp.maximum(m, mj)
        p = jnp.exp((s - mn).astype(jnp.bfloat16))
        pv = jnp.dot(p, vaug(j), preferred_element_type=jnp.float32)
        return mn, acc * jnp.exp(m - mn) + pv

    def step(j, carry):
        return update(scores(j), j, carry)

    m0 = jnp.full((QBLK, 1), -1e30, jnp.float32)
    a0 = jnp.zeros((QBLK, 2 * DH), jnp.float32)
    nkb = iq * (QBLK // KBLK)
    m, acc = jax.lax.fori_loop(0, nkb, step, (m0, a0))

    sd = scores(nkb)
    qpos = iq * QBLK + jax.lax.broadcasted_iota(jnp.int32, (QBLK, KBLK), 0)
    kloc = nkb * KBLK + jax.lax.broadcasted_iota(jnp.int32, (QBLK, KBLK), 1)
    sd = jnp.where(qpos >= kloc, sd, jnp.float32(-1e9))
    m, acc = update(sd, nkb, (m, acc))

    o_ref[0] = acc[:, :DH] / acc[:, DH:DH + 1]


def _attn_call(q3, k3, v3):
    return pl.pallas_call(
        _attn_body,
        grid=(B, H, NQB),
        in_specs=[
            pl.BlockSpec((1, QBLK, DH), lambda b, h, iq: (h, b * NQB + iq, 0)),
            pl.BlockSpec((1, S, DH), lambda b, h, iq: (h // 2, b, 0)),
            pl.BlockSpec((1, S, DH), lambda b, h, iq: (h // 2, b, 0)),
        ],
        out_specs=pl.BlockSpec((1, QBLK, DH),
                               lambda b, h, iq: (h, b * NQB + iq, 0)),
        out_shape=jax.ShapeDtypeStruct((H, T, DH), jnp.float32),
    )(q3, k3, v3)


def _oproj_body(o_ref, wo_ref, res_ref, ln_ref, wg_ref,
                h1_ref, xn2_ref, logits_ref, route_ref):
    h1 = res_ref[...] + jnp.dot(o_ref[...].astype(jnp.bfloat16),
                                wo_ref[...].astype(jnp.bfloat16),
                                preferred_element_type=jnp.float32)
    h1_ref[...] = h1
    ms = jnp.mean(h1 * h1, axis=-1, keepdims=True)
    xn2 = h1 * jax.lax.rsqrt(ms + EPS) * ln_ref[...]
    xn2_ref[...] = xn2
    logits = jnp.dot(xn2.astype(jnp.bfloat16),
                     wg_ref[...].astype(jnp.bfloat16),
                     preferred_element_type=jnp.float32)
    logits_ref[...] = logits
    lm = jnp.max(logits, axis=-1, keepdims=True)
    ex = jnp.exp(logits - lm)
    probs = ex / jnp.sum(ex, axis=-1, keepdims=True)
    ioe = jax.lax.broadcasted_iota(jnp.int32, (RB, E), 1)
    m1 = jnp.max(probs, axis=-1, keepdims=True)
    e1 = jnp.min(jnp.where(probs == m1, ioe, E), axis=-1, keepdims=True)
    probs2 = jnp.where(ioe == e1, jnp.float32(-1.0), probs)
    m2 = jnp.max(probs2, axis=-1, keepdims=True)
    e2 = jnp.min(jnp.where(probs2 == m2, ioe, E), axis=-1, keepdims=True)
    denom = m1 + m2
    w1 = m1 / denom
    w2 = m2 / denom
    route_ref[...] = jnp.concatenate(
        [e1.astype(jnp.float32), e2.astype(jnp.float32), w1, w2,
         jnp.zeros((RB, 4), jnp.float32)], axis=1)


def _oproj_call(o, wo, res, ln2_w, wg):
    return pl.pallas_call(
        _oproj_body,
        grid=(NRB,),
        in_specs=[
            pl.BlockSpec((RB, H * DH), lambda i: (i, 0)),
            pl.BlockSpec((H * DH, D), lambda i: (0, 0)),
            pl.BlockSpec((RB, D), lambda i: (i, 0)),
            pl.BlockSpec((1, D), lambda i: (0, 0)),
            pl.BlockSpec((D, E), lambda i: (0, 0)),
        ],
        out_specs=[
            pl.BlockSpec((RB, D), lambda i: (i, 0)),
            pl.BlockSpec((RB, D), lambda i: (i, 0)),
            pl.BlockSpec((RB, E), lambda i: (i, 0)),
            pl.BlockSpec((RB, E), lambda i: (i, 0)),
        ],
        out_shape=[
            jax.ShapeDtypeStruct((T, D), jnp.float32),
            jax.ShapeDtypeStruct((T, D), jnp.float32),
            jax.ShapeDtypeStruct((T, E), jnp.float32),
            jax.ShapeDtypeStruct((T, E), jnp.float32),
        ],
    )(o, wo, res, ln2_w, wg)


def _gmm_body(be_ref, xs_ref, w1_ref, w3_ref, w2_ref, ws_ref, y_ref):
    del be_ref
    xs = xs_ref[...].astype(jnp.bfloat16)
    a = jnp.dot(xs, w1_ref[0].astype(jnp.bfloat16),
                preferred_element_type=jnp.float32)
    b = jnp.dot(xs, w3_ref[0].astype(jnp.bfloat16),
                preferred_element_type=jnp.float32)
    h = (a / (1.0 + jnp.exp(-a))) * b
    y = jnp.dot(h.astype(jnp.bfloat16), w2_ref[0].astype(jnp.bfloat16),
                preferred_element_type=jnp.float32)
    y_ref[...] = y * ws_ref[...]


def _gmm_call(xs, w1, w3, w2, ws, block_expert):
    grid_spec = pltpu.PrefetchScalarGridSpec(
        num_scalar_prefetch=1,
        grid=(NB,),
        in_specs=[
            pl.BlockSpec((BLK, D), lambda i, be: (i, 0)),
            pl.BlockSpec((1, D, F), lambda i, be: (be[i], 0, 0)),
            pl.BlockSpec((1, D, F), lambda i, be: (be[i], 0, 0)),
            pl.BlockSpec((1, F, D), lambda i, be: (be[i], 0, 0)),
            pl.BlockSpec((BLK, 1), lambda i, be: (i, 0)),
        ],
        out_specs=pl.BlockSpec((BLK, D), lambda i, be: (i, 0)),
    )
    return pl.pallas_call(
        _gmm_body,
        grid_spec=grid_spec,
        out_shape=jax.ShapeDtypeStruct((PAD, D), jnp.float32),
    )(block_expert, xs, w1, w3, w2, ws)


_SC_NC, _SC_NS = 2, 16
_SC_NW = _SC_NC * _SC_NS
GCH = (PAD // _SC_NW) // 3
TCH = (T // _SC_NW) // 4


_G_PER_W = PAD // _SC_NW
_G_OFFS = (0, 48, 96, 144, 192, 240, 288)
_G_LENS = (48, 48, 48, 48, 48, 48, 24)
_G_N = 7


def _sc_gather_rows(x_hbm, idx_hbm, out_hbm, ia, ib, ic, ra, rb, rc,
                    gsa, gsb, wsa, wsb):
    wid = lax.axis_index("s") * _SC_NC + lax.axis_index("c")
    base = wid * _G_PER_W
    idxs = (ia, ib, ia, ib, ia, ib, ic)
    rows = (ra, rb, ra, rb, ra, rb, rc)
    gsems = (gsa, gsb, gsa, gsb, gsa, gsb, gsa)
    wsems = (wsa, wsb, wsa, wsb, wsa, wsb, wsa)
    g = [None] * _G_N
    w = [None] * _G_N

    def start(c):
        pltpu.sync_copy(idx_hbm.at[pl.ds(base + _G_OFFS[c], _G_LENS[c])],
                        idxs[c])
        g[c] = pltpu.async_copy(x_hbm.at[idxs[c]], rows[c], gsems[c])

    start(0)
    start(1)
    for c in range(_G_N):
        g[c].wait()
        w[c] = pltpu.async_copy(
            rows[c], out_hbm.at[pl.ds(base + _G_OFFS[c], _G_LENS[c])],
            wsems[c])
        if c + 2 < _G_N:
            w[c].wait()
            start(c + 2)
    w[_G_N - 2].wait()
    w[_G_N - 1].wait()


def _sc_gather_call(xn2, tok_src):
    mesh = plsc.VectorSubcoreMesh(core_axis_name="c", subcore_axis_name="s")
    f = functools.partial(
        pl.kernel, mesh=mesh,
        out_type=jax.ShapeDtypeStruct((PAD, D), jnp.float32),
        scratch_types=[
            pltpu.VMEM((48,), jnp.int32),
            pltpu.VMEM((48,), jnp.int32),
            pltpu.VMEM((24,), jnp.int32),
            pltpu.VMEM((48, D), jnp.float32),
            pltpu.VMEM((48, D), jnp.float32),
            pltpu.VMEM((24, D), jnp.float32),
            pltpu.SemaphoreType.DMA,
            pltpu.SemaphoreType.DMA,
            pltpu.SemaphoreType.DMA,
            pltpu.SemaphoreType.DMA,
        ],
    )(_sc_gather_rows)
    return f(xn2, tok_src)


def _sc_combine(h1_hbm, y_hbm, p1_hbm, p2_hbm, out_hbm,
                i1_v, i2_v, r1_v, r2_v, h_v, sem):
    wid = lax.axis_index("s") * _SC_NC + lax.axis_index("c")
    base = wid * (T // _SC_NW)
    for c in range(4):
        off = base + c * TCH
        pltpu.sync_copy(p1_hbm.at[pl.ds(off, TCH)], i1_v)
        pltpu.sync_copy(p2_hbm.at[pl.ds(off, TCH)], i2_v)
        pltpu.async_copy(y_hbm.at[i1_v], r1_v, sem).wait()
        pltpu.async_copy(y_hbm.at[i2_v], r2_v, sem).wait()
        pltpu.sync_copy(h1_hbm.at[pl.ds(off, TCH)], h_v)

        def add_step(i, _):
            r = i // (D // 16)
            col = (i % (D // 16)) * 16
            h_v[r, pl.ds(col, 16)] = (h_v[r, pl.ds(col, 16)]
                                      + r1_v[r, pl.ds(col, 16)]
                                      + r2_v[r, pl.ds(col, 16)])
            return 0

        lax.fori_loop(0, TCH * (D // 16), add_step, 0, unroll=4)
        pltpu.sync_copy(h_v, out_hbm.at[pl.ds(off, TCH)])


def _sc_combine_call(h1, y, p1, p2):
    mesh = plsc.VectorSubcoreMesh(core_axis_name="c", subcore_axis_name="s")
    f = functools.partial(
        pl.kernel, mesh=mesh,
        out_type=jax.ShapeDtypeStruct((T, D), jnp.float32),
        scratch_types=[
            pltpu.VMEM((TCH,), jnp.int32),
            pltpu.VMEM((TCH,), jnp.int32),
            pltpu.VMEM((TCH, D), jnp.float32),
            pltpu.VMEM((TCH, D), jnp.float32),
            pltpu.VMEM((TCH, D), jnp.float32),
            pltpu.SemaphoreType.DMA,
        ],
    )(_sc_combine)
    return f(h1, y, p1, p2)


_MB = 512
_NMB = T // _MB


def _meta_body(route_ref, pos_ref, be_ref, rank_ref):
    ioe = jax.lax.broadcasted_iota(jnp.int32, (_MB, E), 1).astype(jnp.float32)
    ra = jax.lax.broadcasted_iota(jnp.int32, (_MB, _MB), 0)
    rb = jax.lax.broadcasted_iota(jnp.int32, (_MB, _MB), 1)
    tri = jnp.where(rb < ra, 1.0, 0.0).astype(jnp.float32)

    carry = jnp.zeros((1, E), jnp.float32)
    for blk in range(_NMB):
        r = route_ref[pl.ds(blk * _MB, _MB), :]
        oh0 = (ioe == r[:, 0:1]).astype(jnp.float32)
        oh1 = (ioe == r[:, 1:2]).astype(jnp.float32)
        ohsum = oh0 + oh1
        cum = jnp.dot(tri, ohsum, preferred_element_type=jnp.float32) + carry
        rank0 = jnp.sum(cum * oh0, axis=1, keepdims=True)
        rank1 = jnp.sum((cum + oh0) * oh1, axis=1, keepdims=True)
        rank_ref[pl.ds(blk * _MB, _MB), :] = jnp.concatenate(
            [rank0, rank1], axis=1)
        carry = carry + jnp.sum(ohsum, axis=0, keepdims=True)

    counts = carry
    pc = jnp.floor((counts + (BLK - 1)) * (1.0 / BLK)) * BLK
    ea = jax.lax.broadcasted_iota(jnp.int32, (E, E), 0)
    eb = jax.lax.broadcasted_iota(jnp.int32, (E, E), 1)
    upper = jnp.where(ea < eb, 1.0, 0.0).astype(jnp.float32)
    poff = jnp.dot(pc, upper, preferred_element_type=jnp.float32)

    bstart = poff * (1.0 / BLK)
    ib = jax.lax.broadcasted_iota(jnp.int32, (E, 64), 1).astype(jnp.float32)
    ge = (ib >= jnp.broadcast_to(bstart.reshape(E, 1), (E, 64)))
    be_ref[...] = (jnp.sum(ge.astype(jnp.float32), axis=0, keepdims=True)
                   - 1.0).astype(jnp.int32)

    for blk in range(_NMB):
        r = route_ref[pl.ds(blk * _MB, _MB), :]
        oh0 = (ioe == r[:, 0:1]).astype(jnp.float32)
        oh1 = (ioe == r[:, 1:2]).astype(jnp.float32)
        po0 = jnp.sum(poff * oh0, axis=1, keepdims=True)
        po1 = jnp.sum(poff * oh1, axis=1, keepdims=True)
        rk = rank_ref[pl.ds(blk * _MB, _MB), :]
        pos = jnp.concatenate([po0 + rk[:, 0:1], po1 + rk[:, 1:2]], axis=1)
        pos_ref[pl.ds(blk * _MB, _MB), :] = pos.astype(jnp.int32)


def _meta_call(route):
    return pl.pallas_call(
        _meta_body,
        grid=(1,),
        in_specs=[pl.BlockSpec((T, E), lambda i: (0, 0))],
        out_specs=[
            pl.BlockSpec((T, 2), lambda i: (0, 0)),
            pl.BlockSpec((1, 64), lambda i: (0, 0)),
        ],
        out_shape=[
            jax.ShapeDtypeStruct((T, 2), jnp.int32),
            jax.ShapeDtypeStruct((1, 64), jnp.int32),
        ],
        scratch_shapes=[pltpu.VMEM((T, 2), jnp.float32)],
    )(route)


_SCH = 128
_NSCH = A // _SCH


def _sc_scatter_body(pf_hbm, wf_hbm, tf_hbm, zi_hbm, zf_hbm,
                     tok_hbm, ws_hbm, pf_v, wf_v, tf_v, sem_t, sem_w):
    wid = lax.axis_index("s") * _SC_NC + lax.axis_index("c")

    @pl.when(wid == 0)
    def _():
        pltpu.sync_copy(pf_hbm, pf_v)
        pltpu.sync_copy(wf_hbm, wf_v)
        pltpu.sync_copy(tf_hbm, tf_v)
        pltpu.sync_copy(zi_hbm, tok_hbm)
        pltpu.sync_copy(zf_hbm, ws_hbm)

        def fire(j, _):
            pltpu.async_copy(tf_v.at[j], tok_hbm.at[pf_v.at[j]], sem_t)
            pltpu.async_copy(wf_v.at[j], ws_hbm.at[pf_v.at[j]], sem_w)
            return 0

        lax.fori_loop(0, _NSCH, fire, 0)

        def drain(j, _):
            pltpu.make_async_copy(tf_v.at[j], tok_hbm.at[pf_v.at[j]],
                                  sem_t).wait()
            pltpu.make_async_copy(wf_v.at[j], ws_hbm.at[pf_v.at[j]],
                                  sem_w).wait()
            return 0

        lax.fori_loop(0, _NSCH, drain, 0)


def _sc_scatter_call(pflat, wflat, tflat, zi, zf):
    mesh = plsc.VectorSubcoreMesh(core_axis_name="c", subcore_axis_name="s")
    f = functools.partial(
        pl.kernel, mesh=mesh,
        out_type=[
            jax.ShapeDtypeStruct((PAD,), jnp.int32),
            jax.ShapeDtypeStruct((PAD,), jnp.float32),
        ],
        scratch_types=[
            pltpu.VMEM((_NSCH, _SCH), jnp.int32),
            pltpu.VMEM((_NSCH, _SCH), jnp.float32),
            pltpu.VMEM((_NSCH, _SCH), jnp.int32),
            pltpu.SemaphoreType.DMA,
            pltpu.SemaphoreType.DMA,
        ],
    )(_sc_scatter_body)
    return f(pflat.reshape(_NSCH, _SCH), wflat.reshape(_NSCH, _SCH),
             tflat.reshape(_NSCH, _SCH), zi, zf)


def kernel(hidden_states, position_ids, gate_logits, ln1_w, ln2_w,
           Wq, Wk, Wv, Wo, Wg, w1, w3, w2):
    x2d = hidden_states.reshape(T, D)
    wqkv = jnp.concatenate([Wq, Wk, Wv], axis=1)

    inv = 1.0 / (THETA ** (jnp.arange(0, DH, 2, dtype=jnp.float32) / DH))
    ang = jnp.arange(S, dtype=jnp.float32)[:, None] * inv[None, :]
    cos_t = jnp.concatenate([jnp.cos(ang), jnp.cos(ang)], axis=1)
    sin_t = jnp.concatenate([jnp.sin(ang), jnp.sin(ang)], axis=1)

    qkv = _qkv_call(x2d, wqkv, ln1_w.reshape(1, D), cos_t, sin_t)
    q3 = qkv[:, :H * DH].reshape(T, H, DH).transpose(1, 0, 2)
    k3 = qkv[:, H * DH:(H + KV) * DH].reshape(T, KV, DH).transpose(1, 0, 2)
    v3 = qkv[:, (H + KV) * DH:].reshape(T, KV, DH).transpose(1, 0, 2)
    o3 = _attn_call(q3, k3, v3)
    o = o3.transpose(1, 0, 2).reshape(T, H * DH)
    h1, xn2, logits, route = _oproj_call(o, Wo, x2d, ln2_w.reshape(1, D), Wg)

    pos2, be64 = _meta_call(route)
    pflat = pos2.reshape(A)
    wflat = route[:, 2:4].reshape(A)
    tflat = jnp.arange(A, dtype=jnp.int32) // TOPK
    tok_src, ws = _sc_scatter_call(pflat, wflat, tflat,
                                   jnp.zeros((PAD,), jnp.int32),
                                   jnp.zeros((PAD,), jnp.float32))
    block_expert = be64[0, :NB]

    xs = _sc_gather_call(xn2, tok_src)
    y = _gmm_call(xs, w1, w3, w2, ws.reshape(PAD, 1), block_expert)

    out2d = _sc_combine_call(h1, y, pos2[:, 0], pos2[:, 1])

    out = out2d.reshape(B, S, D)
    new_gate = gate_logits.at[0].set(logits)
    return (out, position_ids, new_gate)


# speedup vs baseline: 1.3173x; 1.3173x over previous
import functools
import math

import jax
import jax.numpy as jnp
from jax import lax
from jax.experimental import pallas as pl
from jax.experimental.pallas import tpu as pltpu
from jax.experimental.pallas import tpu_sc as plsc

B, S, D = 2, 2048, 1024
H, KV, DH = 16, 8, 64
E, TOPK, F = 8, 2, 2048
THETA, EPS = 10000.0, 1e-6
T = B * S
A = T * TOPK
BLK = 256
NB = (A // BLK) + E - 1
PAD = NB * BLK
RB = 512
NRB = T // RB

HALF = DH // 2
_LOG_THETA = math.log(THETA)


def _qkv_body(x_ref, w_ref, ln_ref, cos_ref, sin_ref, out_ref):
    x = x_ref[...]
    ms = jnp.mean(x * x, axis=-1, keepdims=True)
    xn = x * jax.lax.rsqrt(ms + EPS) * ln_ref[...]
    qkv = jnp.dot(xn.astype(jnp.bfloat16), w_ref[...].astype(jnp.bfloat16),
                  preferred_element_type=jnp.float32)

    QK = (H + KV) * DH
    NH = QK // DH
    qk = qkv[:, :QK]
    cosv = jnp.concatenate([cos_ref[...]] * NH, axis=1)
    sinv = jnp.concatenate([sin_ref[...]] * NH, axis=1)
    col = jax.lax.broadcasted_iota(jnp.int32, (RB, QK), 1)
    rolled_p = jnp.concatenate([qk[:, HALF:], qk[:, :HALF]], axis=1)
    rolled_m = jnp.concatenate([qk[:, -HALF:], qk[:, :-HALF]], axis=1)
    first_half = (col % DH) < HALF
    rh = jnp.where(first_half, -rolled_p, rolled_m)
    qk_rot = qk * cosv + rh * sinv
    out_ref[...] = jnp.concatenate([qk_rot, qkv[:, QK:]], axis=1)


def _qkv_call(x2d, wqkv, ln1_w, cos_t, sin_t):
    nsb = S // RB
    return pl.pallas_call(
        _qkv_body,
        grid=(NRB,),
        in_specs=[
            pl.BlockSpec((RB, D), lambda i: (i, 0)),
            pl.BlockSpec((D, (H + 2 * KV) * DH), lambda i: (0, 0)),
            pl.BlockSpec((1, D), lambda i: (0, 0)),
            pl.BlockSpec((RB, DH), lambda i: (i % nsb, 0)),
            pl.BlockSpec((RB, DH), lambda i: (i % nsb, 0)),
        ],
        out_specs=pl.BlockSpec((RB, (H + 2 * KV) * DH), lambda i: (i, 0)),
        out_shape=jax.ShapeDtypeStruct((T, (H + 2 * KV) * DH), jnp.float32),
    )(x2d, wqkv, ln1_w, cos_t, sin_t)


QBLK = 512
NQB = S // QBLK
KBLK = 512


def _attn_body(q_ref, k_ref, v_ref, o_ref):
    iq = pl.program_id(2)
    q = q_ref[0].astype(jnp.bfloat16)
    lane = jax.lax.broadcasted_iota(jnp.int32, (KBLK, 2 * DH), 1)
    is_v = lane < DH
    is_one = lane == DH

    def vaug(j):
        v = v_ref[0, pl.ds(j * KBLK, KBLK), :]
        vb = jnp.concatenate([v, v], axis=1)
        return jnp.where(is_v, vb, jnp.where(is_one, 1.0, 0.0)
                         ).astype(jnp.bfloat16)

    def scores(j):
        k = k_ref[0, pl.ds(j * KBLK, KBLK), :].astype(jnp.bfloat16)
        s = jax.lax.dot_general(q, k, (((1,), (1,)), ((), ())),
                                preferred_element_type=jnp.float32)
        return s * (1.0 / 8.0)

    def update(s, j, carry):
        m, acc = carry
        mj = jnp.max(s, axis=-1, keepdims=True)
        mn = jnp.maximum(m, mj)
        p = jnp.exp((s - mn).astype(jnp.bfloat16))
        pv = jnp.dot(p, vaug(j), preferred_element_type=jnp.float32)
        return mn, acc * jnp.exp(m - mn) + pv

    def step(j, carry):
        return update(scores(j), j, carry)

    m0 = jnp.full((QBLK, 1), -1e30, jnp.float32)
    a0 = jnp.zeros((QBLK, 2 * DH), jnp.float32)
    nkb = iq * (QBLK // KBLK)
    m, acc = jax.lax.fori_loop(0, nkb, step, (m0, a0))

    sd = scores(nkb)
    qpos = iq * QBLK + jax.lax.broadcasted_iota(jnp.int32, (QBLK, KBLK), 0)
    kloc = nkb * KBLK + jax.lax.broadcasted_iota(jnp.int32, (QBLK, KBLK), 1)
    sd = jnp.where(qpos >= kloc, sd, jnp.float32(-1e9))
    m, acc = update(sd, nkb, (m, acc))

    o_ref[0] = acc[:, :DH] / acc[:, DH:DH + 1]


def _attn_call(q3, k3, v3):
    return pl.pallas_call(
        _attn_body,
        grid=(B, H, NQB),
        in_specs=[
            pl.BlockSpec((1, QBLK, DH), lambda b, h, iq: (h, b * NQB + iq, 0)),
            pl.BlockSpec((1, S, DH), lambda b, h, iq: (h // 2, b, 0)),
            pl.BlockSpec((1, S, DH), lambda b, h, iq: (h // 2, b, 0)),
        ],
        out_specs=pl.BlockSpec((1, QBLK, DH),
                               lambda b, h, iq: (h, b * NQB + iq, 0)),
        out_shape=jax.ShapeDtypeStruct((H, T, DH), jnp.float32),
    )(q3, k3, v3)


def _oproj_body(o_ref, wo_ref, res_ref, ln_ref, wg_ref,
                h1_ref, xn2_ref, logits_ref, route_ref):
    h1 = res_ref[...] + jnp.dot(o_ref[...].astype(jnp.bfloat16),
                                wo_ref[...].astype(jnp.bfloat16),
                                preferred_element_type=jnp.float32)
    h1_ref[...] = h1
    ms = jnp.mean(h1 * h1, axis=-1, keepdims=True)
    xn2 = h1 * jax.lax.rsqrt(ms + EPS) * ln_ref[...]
    xn2_ref[...] = xn2
    logits = jnp.dot(xn2.astype(jnp.bfloat16),
                     wg_ref[...].astype(jnp.bfloat16),
                     preferred_element_type=jnp.float32)
    logits_ref[...] = logits
    lm = jnp.max(logits, axis=-1, keepdims=True)
    ex = jnp.exp(logits - lm)
    probs = ex / jnp.sum(ex, axis=-1, keepdims=True)
    ioe = jax.lax.broadcasted_iota(jnp.int32, (RB, E), 1)
    m1 = jnp.max(probs, axis=-1, keepdims=True)
    e1 = jnp.min(jnp.where(probs == m1, ioe, E), axis=-1, keepdims=True)
    probs2 = jnp.where(ioe == e1, jnp.float32(-1.0), probs)
    m2 = jnp.max(probs2, axis=-1, keepdims=True)
    e2 = jnp.min(jnp.where(probs2 == m2, ioe, E), axis=-1, keepdims=True)
    denom = m1 + m2
    w1 = m1 / denom
    w2 = m2 / denom
    route_ref[...] = jnp.concatenate(
        [e1.astype(jnp.float32), e2.astype(jnp.float32), w1, w2,
         jnp.zeros((RB, 4), jnp.float32)], axis=1)


def _oproj_call(o, wo, res, ln2_w, wg):
    return pl.pallas_call(
        _oproj_body,
        grid=(NRB,),
        in_specs=[
            pl.BlockSpec((RB, H * DH), lambda i: (i, 0)),
            pl.BlockSpec((H * DH, D), lambda i: (0, 0)),
            pl.BlockSpec((RB, D), lambda i: (i, 0)),
            pl.BlockSpec((1, D), lambda i: (0, 0)),
            pl.BlockSpec((D, E), lambda i: (0, 0)),
        ],
        out_specs=[
            pl.BlockSpec((RB, D), lambda i: (i, 0)),
            pl.BlockSpec((RB, D), lambda i: (i, 0)),
            pl.BlockSpec((RB, E), lambda i: (i, 0)),
            pl.BlockSpec((RB, E), lambda i: (i, 0)),
        ],
        out_shape=[
            jax.ShapeDtypeStruct((T, D), jnp.float32),
            jax.ShapeDtypeStruct((T, D), jnp.float32),
            jax.ShapeDtypeStruct((T, E), jnp.float32),
            jax.ShapeDtypeStruct((T, E), jnp.float32),
        ],
    )(o, wo, res, ln2_w, wg)


def _gmm_body(be_ref, xs_ref, w1_ref, w3_ref, w2_ref, y_ref):
    del be_ref
    xs = xs_ref[...].astype(jnp.bfloat16)
    a = jnp.dot(xs, w1_ref[0].astype(jnp.bfloat16),
                preferred_element_type=jnp.float32)
    b = jnp.dot(xs, w3_ref[0].astype(jnp.bfloat16),
                preferred_element_type=jnp.float32)
    h = (a / (1.0 + jnp.exp(-a))) * b
    y_ref[...] = jnp.dot(h.astype(jnp.bfloat16),
                         w2_ref[0].astype(jnp.bfloat16),
                         preferred_element_type=jnp.float32)


def _gmm_call(xs, w1, w3, w2, block_expert):
    grid_spec = pltpu.PrefetchScalarGridSpec(
        num_scalar_prefetch=1,
        grid=(NB,),
        in_specs=[
            pl.BlockSpec((BLK, D), lambda i, be: (i, 0)),
            pl.BlockSpec((1, D, F), lambda i, be: (be[i], 0, 0)),
            pl.BlockSpec((1, D, F), lambda i, be: (be[i], 0, 0)),
            pl.BlockSpec((1, F, D), lambda i, be: (be[i], 0, 0)),
        ],
        out_specs=pl.BlockSpec((BLK, D), lambda i, be: (i, 0)),
    )
    return pl.pallas_call(
        _gmm_body,
        grid_spec=grid_spec,
        out_shape=jax.ShapeDtypeStruct((PAD, D), jnp.float32),
    )(block_expert, xs, w1, w3, w2)


_SC_NC, _SC_NS = 2, 16
_SC_NW = _SC_NC * _SC_NS
GCH = (PAD // _SC_NW) // 3
TCH = (T // _SC_NW) // 4


_G_PER_W = PAD // _SC_NW
_G_OFFS = (0, 48, 96, 144, 192, 240, 288)
_G_LENS = (48, 48, 48, 48, 48, 48, 24)
_G_N = 7


_DCH = 64


def _sc_dispatch_body(x_hbm, p0_hbm, p1_hbm, out_hbm, i0, i1, xv, s0, s1):
    wid = lax.axis_index("s") * _SC_NC + lax.axis_index("c")
    tbase = wid * (T // _SC_NW)
    for c in range(2):
        off = tbase + c * _DCH
        pltpu.sync_copy(p0_hbm.at[pl.ds(off, _DCH)], i0)
        pltpu.sync_copy(p1_hbm.at[pl.ds(off, _DCH)], i1)
        pltpu.sync_copy(x_hbm.at[pl.ds(off, _DCH)], xv)
        c0 = pltpu.async_copy(xv, out_hbm.at[i0], s0)
        c1 = pltpu.async_copy(xv, out_hbm.at[i1], s1)
        c0.wait()
        c1.wait()


def _sc_dispatch_call(xn2, p0, p1):
    mesh = plsc.VectorSubcoreMesh(core_axis_name="c", subcore_axis_name="s")
    f = functools.partial(
        pl.kernel, mesh=mesh,
        out_type=jax.ShapeDtypeStruct((PAD, D), jnp.float32),
        scratch_types=[
            pltpu.VMEM((_DCH,), jnp.int32),
            pltpu.VMEM((_DCH,), jnp.int32),
            pltpu.VMEM((_DCH, D), jnp.float32),
            pltpu.SemaphoreType.DMA,
            pltpu.SemaphoreType.DMA,
        ],
    )(_sc_dispatch_body)
    return f(xn2, p0, p1)


def _sc_combine(h1_hbm, y_hbm, p1_hbm, p2_hbm, w1_hbm, w2_hbm, out_hbm,
                i1_v, i2_v, w1_v, w2_v, r1_v, r2_v, h_v, sem):
    wid = lax.axis_index("s") * _SC_NC + lax.axis_index("c")
    base = wid * (T // _SC_NW)
    for c in range(4):
        off = base + c * TCH
        pltpu.sync_copy(p1_hbm.at[pl.ds(off, TCH)], i1_v)
        pltpu.sync_copy(p2_hbm.at[pl.ds(off, TCH)], i2_v)
        pltpu.sync_copy(w1_hbm.at[pl.ds(off, TCH)], w1_v)
        pltpu.sync_copy(w2_hbm.at[pl.ds(off, TCH)], w2_v)
        pltpu.async_copy(y_hbm.at[i1_v], r1_v, sem).wait()
        pltpu.async_copy(y_hbm.at[i2_v], r2_v, sem).wait()
        pltpu.sync_copy(h1_hbm.at[pl.ds(off, TCH)], h_v)

        def row_step(r, _):
            wb1 = w1_v[r, :]
            wb2 = w2_v[r, :]

            def col_step(j, _):
                col = j * 16
                h_v[r, pl.ds(col, 16)] = (h_v[r, pl.ds(col, 16)]
                                          + wb1 * r1_v[r, pl.ds(col, 16)]
                                          + wb2 * r2_v[r, pl.ds(col, 16)])
                return 0

            lax.fori_loop(0, D // 16, col_step, 0, unroll=4)
            return 0

        lax.fori_loop(0, TCH, row_step, 0)
        pltpu.sync_copy(h_v, out_hbm.at[pl.ds(off, TCH)])


def _sc_combine_call(h1, y, p1, p2, w1c, w2c):
    w1x = jnp.broadcast_to(w1c[:, None], (T, 16))
    w2x = jnp.broadcast_to(w2c[:, None], (T, 16))
    mesh = plsc.VectorSubcoreMesh(core_axis_name="c", subcore_axis_name="s")
    f = functools.partial(
        pl.kernel, mesh=mesh,
        out_type=jax.ShapeDtypeStruct((T, D), jnp.float32),
        scratch_types=[
            pltpu.VMEM((TCH,), jnp.int32),
            pltpu.VMEM((TCH,), jnp.int32),
            pltpu.VMEM((TCH, 16), jnp.float32),
            pltpu.VMEM((TCH, 16), jnp.float32),
            pltpu.VMEM((TCH, D), jnp.float32),
            pltpu.VMEM((TCH, D), jnp.float32),
            pltpu.VMEM((TCH, D), jnp.float32),
            pltpu.SemaphoreType.DMA,
        ],
    )(_sc_combine)
    return f(h1, y, p1, p2, w1x, w2x)


_MB = 512
_NMB = T // _MB


def _meta_body(route_ref, pos_ref, be_ref, rank_ref):
    ioe = jax.lax.broadcasted_iota(jnp.int32, (_MB, E), 1).astype(jnp.float32)
    ra = jax.lax.broadcasted_iota(jnp.int32, (_MB, _MB), 0)
    rb = jax.lax.broadcasted_iota(jnp.int32, (_MB, _MB), 1)
    tri = jnp.where(rb < ra, 1.0, 0.0).astype(jnp.float32)

    carry = jnp.zeros((1, E), jnp.float32)
    for blk in range(_NMB):
        r = route_ref[pl.ds(blk * _MB, _MB), :]
        oh0 = (ioe == r[:, 0:1]).astype(jnp.float32)
        oh1 = (ioe == r[:, 1:2]).astype(jnp.float32)
        ohsum = oh0 + oh1
        cum = jnp.dot(tri, ohsum, preferred_element_type=jnp.float32) + carry
        rank0 = jnp.sum(cum * oh0, axis=1, keepdims=True)
        rank1 = jnp.sum((cum + oh0) * oh1, axis=1, keepdims=True)
        rank_ref[pl.ds(blk * _MB, _MB), :] = jnp.concatenate(
            [rank0, rank1], axis=1)
        carry = carry + jnp.sum(ohsum, axis=0, keepdims=True)

    counts = carry
    pc = jnp.floor((counts + (BLK - 1)) * (1.0 / BLK)) * BLK
    ea = jax.lax.broadcasted_iota(jnp.int32, (E, E), 0)
    eb = jax.lax.broadcasted_iota(jnp.int32, (E, E), 1)
    upper = jnp.where(ea < eb, 1.0, 0.0).astype(jnp.float32)
    poff = jnp.dot(pc, upper, preferred_element_type=jnp.float32)

    bstart = poff * (1.0 / BLK)
    ib = jax.lax.broadcasted_iota(jnp.int32, (E, 64), 1).astype(jnp.float32)
    ge = (ib >= jnp.broadcast_to(bstart.reshape(E, 1), (E, 64)))
    be_ref[...] = (jnp.sum(ge.astype(jnp.float32), axis=0, keepdims=True)
                   - 1.0).astype(jnp.int32)

    for blk in range(_NMB):
        r = route_ref[pl.ds(blk * _MB, _MB), :]
        oh0 = (ioe == r[:, 0:1]).astype(jnp.float32)
        oh1 = (ioe == r[:, 1:2]).astype(jnp.float32)
        po0 = jnp.sum(poff * oh0, axis=1, keepdims=True)
        po1 = jnp.sum(poff * oh1, axis=1, keepdims=True)
        rk = rank_ref[pl.ds(blk * _MB, _MB), :]
        pos = jnp.concatenate([po0 + rk[:, 0:1], po1 + rk[:, 1:2]], axis=1)
        pos_ref[pl.ds(blk * _MB, _MB), :] = pos.astype(jnp.int32)


def _meta_call(route):
    return pl.pallas_call(
        _meta_body,
        grid=(1,),
        in_specs=[pl.BlockSpec((T, E), lambda i: (0, 0))],
        out_specs=[
            pl.BlockSpec((T, 2), lambda i: (0, 0)),
            pl.BlockSpec((1, 64), lambda i: (0, 0)),
        ],
        out_shape=[
            jax.ShapeDtypeStruct((T, 2), jnp.int32),
            jax.ShapeDtypeStruct((1, 64), jnp.int32),
        ],
        scratch_shapes=[pltpu.VMEM((T, 2), jnp.float32)],
    )(route)


def kernel(hidden_states, position_ids, gate_logits, ln1_w, ln2_w,
           Wq, Wk, Wv, Wo, Wg, w1, w3, w2):
    x2d = hidden_states.reshape(T, D)
    wqkv = jnp.concatenate([Wq, Wk, Wv], axis=1)

    inv = 1.0 / (THETA ** (jnp.arange(0, DH, 2, dtype=jnp.float32) / DH))
    ang = jnp.arange(S, dtype=jnp.float32)[:, None] * inv[None, :]
    cos_t = jnp.concatenate([jnp.cos(ang), jnp.cos(ang)], axis=1)
    sin_t = jnp.concatenate([jnp.sin(ang), jnp.sin(ang)], axis=1)

    qkv = _qkv_call(x2d, wqkv, ln1_w.reshape(1, D), cos_t, sin_t)
    q3 = qkv[:, :H * DH].reshape(T, H, DH).transpose(1, 0, 2)
    k3 = qkv[:, H * DH:(H + KV) * DH].reshape(T, KV, DH).transpose(1, 0, 2)
    v3 = qkv[:, (H + KV) * DH:].reshape(T, KV, DH).transpose(1, 0, 2)
    o3 = _attn_call(q3, k3, v3)
    o = o3.transpose(1, 0, 2).reshape(T, H * DH)
    h1, xn2, logits, route = _oproj_call(o, Wo, x2d, ln2_w.reshape(1, D), Wg)

    pos2, be64 = _meta_call(route)
    block_expert = be64[0, :NB]
    p0 = pos2[:, 0]
    p1 = pos2[:, 1]

    xs = _sc_dispatch_call(xn2, p0, p1)
    y = _gmm_call(xs, w1, w3, w2, block_expert)

    out2d = _sc_combine_call(h1, y, p0, p1, route[:, 2], route[:, 3])

    out = out2d.reshape(B, S, D)
    new_gate = gate_logits.at[0].set(logits)
    return (out, position_ids, new_gate)


# speedup vs baseline: 1.3176x; 1.0003x over previous
import functools
import math

import jax
import jax.numpy as jnp
from jax import lax
from jax.experimental import pallas as pl
from jax.experimental.pallas import tpu as pltpu
from jax.experimental.pallas import tpu_sc as plsc

B, S, D = 2, 2048, 1024
H, KV, DH = 16, 8, 64
E, TOPK, F = 8, 2, 2048
THETA, EPS = 10000.0, 1e-6
T = B * S
A = T * TOPK
BLK = 256
NB = (A // BLK) + E - 1
PAD = NB * BLK
RB = 512
NRB = T // RB

HALF = DH // 2
_LOG_THETA = math.log(THETA)


def _qkv_body(x_ref, w_ref, ln_ref, cos_ref, sin_ref, out_ref):
    x = x_ref[...]
    ms = jnp.mean(x * x, axis=-1, keepdims=True)
    xn = x * jax.lax.rsqrt(ms + EPS) * ln_ref[...]
    qkv = jnp.dot(xn.astype(jnp.bfloat16), w_ref[...].astype(jnp.bfloat16),
                  preferred_element_type=jnp.float32)

    QK = (H + KV) * DH
    NH = QK // DH
    qk = qkv[:, :QK]
    cosv = jnp.concatenate([cos_ref[...]] * NH, axis=1)
    sinv = jnp.concatenate([sin_ref[...]] * NH, axis=1)
    col = jax.lax.broadcasted_iota(jnp.int32, (RB, QK), 1)
    rolled_p = jnp.concatenate([qk[:, HALF:], qk[:, :HALF]], axis=1)
    rolled_m = jnp.concatenate([qk[:, -HALF:], qk[:, :-HALF]], axis=1)
    first_half = (col % DH) < HALF
    rh = jnp.where(first_half, -rolled_p, rolled_m)
    qk_rot = qk * cosv + rh * sinv
    out_ref[...] = jnp.concatenate([qk_rot, qkv[:, QK:]], axis=1)


def _qkv_call(x2d, wqkv, ln1_w, cos_t, sin_t):
    nsb = S // RB
    return pl.pallas_call(
        _qkv_body,
        grid=(NRB,),
        in_specs=[
            pl.BlockSpec((RB, D), lambda i: (i, 0)),
            pl.BlockSpec((D, (H + 2 * KV) * DH), lambda i: (0, 0)),
            pl.BlockSpec((1, D), lambda i: (0, 0)),
            pl.BlockSpec((RB, DH), lambda i: (i % nsb, 0)),
            pl.BlockSpec((RB, DH), lambda i: (i % nsb, 0)),
        ],
        out_specs=pl.BlockSpec((RB, (H + 2 * KV) * DH), lambda i: (i, 0)),
        out_shape=jax.ShapeDtypeStruct((T, (H + 2 * KV) * DH), jnp.float32),
    )(x2d, wqkv, ln1_w, cos_t, sin_t)


QBLK = 512
NQB = S // QBLK
KBLK = 512


def _attn_body(q_ref, k_ref, v_ref, o_ref):
    iq = pl.program_id(2)
    q = q_ref[0].astype(jnp.bfloat16)
    lane = jax.lax.broadcasted_iota(jnp.int32, (KBLK, 2 * DH), 1)
    is_v = lane < DH
    is_one = lane == DH

    def vaug(j):
        v = v_ref[0, pl.ds(j * KBLK, KBLK), :]
        vb = jnp.concatenate([v, v], axis=1)
        return jnp.where(is_v, vb, jnp.where(is_one, 1.0, 0.0)
                         ).astype(jnp.bfloat16)

    def scores(j):
        k = k_ref[0, pl.ds(j * KBLK, KBLK), :].astype(jnp.bfloat16)
        s = jax.lax.dot_general(q, k, (((1,), (1,)), ((), ())),
                                preferred_element_type=jnp.float32)
        return s * (1.0 / 8.0)

    def update(s, j, carry):
        m, acc = carry
        mj = jnp.max(s, axis=-1, keepdims=True)
        mn = jnp.maximum(m, mj)
        p = jnp.exp((s - mn).astype(jnp.bfloat16))
        pv = jnp.dot(p, vaug(j), preferred_element_type=jnp.float32)
        return mn, acc * jnp.exp(m - mn) + pv

    def step(j, carry):
        return update(scores(j), j, carry)

    m0 = jnp.full((QBLK, 1), -1e30, jnp.float32)
    a0 = jnp.zeros((QBLK, 2 * DH), jnp.float32)
    nkb = iq * (QBLK // KBLK)
    m, acc = jax.lax.fori_loop(0, nkb, step, (m0, a0))

    sd = scores(nkb)
    qpos = iq * QBLK + jax.lax.broadcasted_iota(jnp.int32, (QBLK, KBLK), 0)
    kloc = nkb * KBLK + jax.lax.broadcasted_iota(jnp.int32, (QBLK, KBLK), 1)
    sd = jnp.where(qpos >= kloc, sd, jnp.float32(-1e9))
    m, acc = update(sd, nkb, (m, acc))

    o_ref[0] = acc[:, :DH] / acc[:, DH:DH + 1]


def _attn_call(q3, k3, v3):
    return pl.pallas_call(
        _attn_body,
        grid=(B, H, NQB),
        in_specs=[
            pl.BlockSpec((1, QBLK, DH), lambda b, h, iq: (h, b * NQB + iq, 0)),
            pl.BlockSpec((1, S, DH), lambda b, h, iq: (h // 2, b, 0)),
            pl.BlockSpec((1, S, DH), lambda b, h, iq: (h // 2, b, 0)),
        ],
        out_specs=pl.BlockSpec((1, QBLK, DH),
                               lambda b, h, iq: (h, b * NQB + iq, 0)),
        out_shape=jax.ShapeDtypeStruct((H, T, DH), jnp.float32),
    )(q3, k3, v3)


def _oproj_body(o_ref, wo_ref, res_ref, ln_ref, wg_ref,
                h1_ref, xn2_ref, logits_ref, route_ref):
    h1 = res_ref[...] + jnp.dot(o_ref[...].astype(jnp.bfloat16),
                                wo_ref[...].astype(jnp.bfloat16),
                                preferred_element_type=jnp.float32)
    h1_ref[...] = h1
    ms = jnp.mean(h1 * h1, axis=-1, keepdims=True)
    xn2 = h1 * jax.lax.rsqrt(ms + EPS) * ln_ref[...]
    xn2_ref[...] = xn2
    logits = jnp.dot(xn2.astype(jnp.bfloat16),
                     wg_ref[...].astype(jnp.bfloat16),
                     preferred_element_type=jnp.float32)
    logits_ref[...] = logits
    lm = jnp.max(logits, axis=-1, keepdims=True)
    ex = jnp.exp(logits - lm)
    probs = ex / jnp.sum(ex, axis=-1, keepdims=True)
    ioe = jax.lax.broadcasted_iota(jnp.int32, (RB, E), 1)
    m1 = jnp.max(probs, axis=-1, keepdims=True)
    e1 = jnp.min(jnp.where(probs == m1, ioe, E), axis=-1, keepdims=True)
    probs2 = jnp.where(ioe == e1, jnp.float32(-1.0), probs)
    m2 = jnp.max(probs2, axis=-1, keepdims=True)
    e2 = jnp.min(jnp.where(probs2 == m2, ioe, E), axis=-1, keepdims=True)
    denom = m1 + m2
    w1 = m1 / denom
    w2 = m2 / denom
    route_ref[...] = jnp.concatenate(
        [e1.astype(jnp.float32), e2.astype(jnp.float32), w1, w2,
         jnp.zeros((RB, 4), jnp.float32)], axis=1)


def _oproj_call(o, wo, res, ln2_w, wg):
    return pl.pallas_call(
        _oproj_body,
        grid=(NRB,),
        in_specs=[
            pl.BlockSpec((RB, H * DH), lambda i: (i, 0)),
            pl.BlockSpec((H * DH, D), lambda i: (0, 0)),
            pl.BlockSpec((RB, D), lambda i: (i, 0)),
            pl.BlockSpec((1, D), lambda i: (0, 0)),
            pl.BlockSpec((D, E), lambda i: (0, 0)),
        ],
        out_specs=[
            pl.BlockSpec((RB, D), lambda i: (i, 0)),
            pl.BlockSpec((RB, D), lambda i: (i, 0)),
            pl.BlockSpec((RB, E), lambda i: (i, 0)),
            pl.BlockSpec((RB, E), lambda i: (i, 0)),
        ],
        out_shape=[
            jax.ShapeDtypeStruct((T, D), jnp.float32),
            jax.ShapeDtypeStruct((T, D), jnp.float32),
            jax.ShapeDtypeStruct((T, E), jnp.float32),
            jax.ShapeDtypeStruct((T, E), jnp.float32),
        ],
    )(o, wo, res, ln2_w, wg)


def _gmm_body(be_ref, xs_ref, w1_ref, w3_ref, w2_ref, y_ref):
    del be_ref
    xs = xs_ref[...].astype(jnp.bfloat16)
    a = jnp.dot(xs, w1_ref[0].astype(jnp.bfloat16),
                preferred_element_type=jnp.float32)
    b = jnp.dot(xs, w3_ref[0].astype(jnp.bfloat16),
                preferred_element_type=jnp.float32)
    h = (a / (1.0 + jnp.exp(-a))) * b
    y_ref[...] = jnp.dot(h.astype(jnp.bfloat16),
                         w2_ref[0].astype(jnp.bfloat16),
                         preferred_element_type=jnp.float32)


def _gmm_call(xs, w1, w3, w2, block_expert):
    grid_spec = pltpu.PrefetchScalarGridSpec(
        num_scalar_prefetch=1,
        grid=(NB,),
        in_specs=[
            pl.BlockSpec((BLK, D), lambda i, be: (i, 0)),
            pl.BlockSpec((1, D, F), lambda i, be: (be[i], 0, 0)),
            pl.BlockSpec((1, D, F), lambda i, be: (be[i], 0, 0)),
            pl.BlockSpec((1, F, D), lambda i, be: (be[i], 0, 0)),
        ],
        out_specs=pl.BlockSpec((BLK, D), lambda i, be: (i, 0)),
    )
    return pl.pallas_call(
        _gmm_body,
        grid_spec=grid_spec,
        out_shape=jax.ShapeDtypeStruct((PAD, D), jnp.float32),
    )(block_expert, xs, w1, w3, w2)


_SC_NC, _SC_NS = 2, 16
_SC_NW = _SC_NC * _SC_NS
TCH = (T // _SC_NW) // 4
_DCH = 64


def _sc_dispatch_body(x_hbm, p0_hbm, p1_hbm, out_hbm, i0, i1, xv, s0, s1):
    wid = lax.axis_index("s") * _SC_NC + lax.axis_index("c")
    tbase = wid * (T // _SC_NW)
    for c in range(2):
        off = tbase + c * _DCH
        pltpu.sync_copy(p0_hbm.at[pl.ds(off, _DCH)], i0)
        pltpu.sync_copy(p1_hbm.at[pl.ds(off, _DCH)], i1)
        pltpu.sync_copy(x_hbm.at[pl.ds(off, _DCH)], xv)
        c0 = pltpu.async_copy(xv, out_hbm.at[i0], s0)
        c1 = pltpu.async_copy(xv, out_hbm.at[i1], s1)
        c0.wait()
        c1.wait()


def _sc_dispatch_call(xn2, p0, p1):
    mesh = plsc.VectorSubcoreMesh(core_axis_name="c", subcore_axis_name="s")
    f = functools.partial(
        pl.kernel, mesh=mesh,
        out_type=jax.ShapeDtypeStruct((PAD, D), jnp.float32),
        scratch_types=[
            pltpu.VMEM((_DCH,), jnp.int32),
            pltpu.VMEM((_DCH,), jnp.int32),
            pltpu.VMEM((_DCH, D), jnp.float32),
            pltpu.SemaphoreType.DMA,
            pltpu.SemaphoreType.DMA,
        ],
    )(_sc_dispatch_body)
    return f(xn2, p0, p1)


def _sc_combine(h1_hbm, y_hbm, p1_hbm, p2_hbm, w1_hbm, w2_hbm, out_hbm,
                i1_v, i2_v, w1_v, w2_v, r1_v, r2_v, h_v, sem):
    wid = lax.axis_index("s") * _SC_NC + lax.axis_index("c")
    base = wid * (T // _SC_NW)
    for c in range(4):
        off = base + c * TCH
        pltpu.sync_copy(p1_hbm.at[pl.ds(off, TCH)], i1_v)
        pltpu.sync_copy(p2_hbm.at[pl.ds(off, TCH)], i2_v)
        pltpu.sync_copy(w1_hbm.at[pl.ds(off, TCH)], w1_v)
        pltpu.sync_copy(w2_hbm.at[pl.ds(off, TCH)], w2_v)
        pltpu.async_copy(y_hbm.at[i1_v], r1_v, sem).wait()
        pltpu.async_copy(y_hbm.at[i2_v], r2_v, sem).wait()
        pltpu.sync_copy(h1_hbm.at[pl.ds(off, TCH)], h_v)

        def row_step(r, _):
            wb1 = w1_v[r, :]
            wb2 = w2_v[r, :]

            def col_step(j, _):
                col = j * 16
                h_v[r, pl.ds(col, 16)] = (h_v[r, pl.ds(col, 16)]
                                          + wb1 * r1_v[r, pl.ds(col, 16)]
                                          + wb2 * r2_v[r, pl.ds(col, 16)])
                return 0

            lax.fori_loop(0, D // 16, col_step, 0, unroll=4)
            return 0

        lax.fori_loop(0, TCH, row_step, 0)
        pltpu.sync_copy(h_v, out_hbm.at[pl.ds(off, TCH)])


def _sc_combine_call(h1, y, p1, p2, w1c, w2c):
    w1x = jnp.broadcast_to(w1c[:, None], (T, 16))
    w2x = jnp.broadcast_to(w2c[:, None], (T, 16))
    mesh = plsc.VectorSubcoreMesh(core_axis_name="c", subcore_axis_name="s")
    f = functools.partial(
        pl.kernel, mesh=mesh,
        out_type=jax.ShapeDtypeStruct((T, D), jnp.float32),
        scratch_types=[
            pltpu.VMEM((TCH,), jnp.int32),
            pltpu.VMEM((TCH,), jnp.int32),
            pltpu.VMEM((TCH, 16), jnp.float32),
            pltpu.VMEM((TCH, 16), jnp.float32),
            pltpu.VMEM((TCH, D), jnp.float32),
            pltpu.VMEM((TCH, D), jnp.float32),
            pltpu.VMEM((TCH, D), jnp.float32),
            pltpu.SemaphoreType.DMA,
        ],
    )(_sc_combine)
    return f(h1, y, p1, p2, w1x, w2x)


_MB = 512
_NMB = T // _MB


def _meta_body(route_ref, pos_ref, be_ref, rank_ref):
    ioe = jax.lax.broadcasted_iota(jnp.int32, (_MB, E), 1).astype(jnp.float32)
    ra = jax.lax.broadcasted_iota(jnp.int32, (_MB, _MB), 0)
    rb = jax.lax.broadcasted_iota(jnp.int32, (_MB, _MB), 1)
    tri = jnp.where(rb < ra, 1.0, 0.0).astype(jnp.float32)

    carry = jnp.zeros((1, E), jnp.float32)
    for blk in range(_NMB):
        r = route_ref[pl.ds(blk * _MB, _MB), :]
        oh0 = (ioe == r[:, 0:1]).astype(jnp.float32)
        oh1 = (ioe == r[:, 1:2]).astype(jnp.float32)
        ohsum = oh0 + oh1
        cum = jnp.dot(tri, ohsum, preferred_element_type=jnp.float32) + carry
        rank0 = jnp.sum(cum * oh0, axis=1, keepdims=True)
        rank1 = jnp.sum((cum + oh0) * oh1, axis=1, keepdims=True)
        rank_ref[pl.ds(blk * _MB, _MB), :] = jnp.concatenate(
            [rank0, rank1], axis=1)
        carry = carry + jnp.sum(ohsum, axis=0, keepdims=True)

    counts = carry
    pc = jnp.floor((counts + (BLK - 1)) * (1.0 / BLK)) * BLK
    ea = jax.lax.broadcasted_iota(jnp.int32, (E, E), 0)
    eb = jax.lax.broadcasted_iota(jnp.int32, (E, E), 1)
    upper = jnp.where(ea < eb, 1.0, 0.0).astype(jnp.float32)
    poff = jnp.dot(pc, upper, preferred_element_type=jnp.float32)

    bstart = poff * (1.0 / BLK)
    ib = jax.lax.broadcasted_iota(jnp.int32, (E, 64), 1).astype(jnp.float32)
    ge = (ib >= jnp.broadcast_to(bstart.reshape(E, 1), (E, 64)))
    be_ref[...] = (jnp.sum(ge.astype(jnp.float32), axis=0, keepdims=True)
                   - 1.0).astype(jnp.int32)

    for blk in range(_NMB):
        r = route_ref[pl.ds(blk * _MB, _MB), :]
        oh0 = (ioe == r[:, 0:1]).astype(jnp.float32)
        oh1 = (ioe == r[:, 1:2]).astype(jnp.float32)
        po0 = jnp.sum(poff * oh0, axis=1, keepdims=True)
        po1 = jnp.sum(poff * oh1, axis=1, keepdims=True)
        rk = rank_ref[pl.ds(blk * _MB, _MB), :]
        pos = jnp.concatenate([po0 + rk[:, 0:1], po1 + rk[:, 1:2]], axis=1)
        pos_ref[pl.ds(blk * _MB, _MB), :] = pos.astype(jnp.int32)


def _meta_call(route):
    return pl.pallas_call(
        _meta_body,
        grid=(1,),
        in_specs=[pl.BlockSpec((T, E), lambda i: (0, 0))],
        out_specs=[
            pl.BlockSpec((T, 2), lambda i: (0, 0)),
            pl.BlockSpec((1, 64), lambda i: (0, 0)),
        ],
        out_shape=[
            jax.ShapeDtypeStruct((T, 2), jnp.int32),
            jax.ShapeDtypeStruct((1, 64), jnp.int32),
        ],
        scratch_shapes=[pltpu.VMEM((T, 2), jnp.float32)],
    )(route)


def kernel(hidden_states, position_ids, gate_logits, ln1_w, ln2_w,
           Wq, Wk, Wv, Wo, Wg, w1, w3, w2):
    x2d = hidden_states.reshape(T, D)
    wqkv = jnp.concatenate([Wq, Wk, Wv], axis=1)

    inv = 1.0 / (THETA ** (jnp.arange(0, DH, 2, dtype=jnp.float32) / DH))
    ang = jnp.arange(S, dtype=jnp.float32)[:, None] * inv[None, :]
    cos_t = jnp.concatenate([jnp.cos(ang), jnp.cos(ang)], axis=1)
    sin_t = jnp.concatenate([jnp.sin(ang), jnp.sin(ang)], axis=1)

    qkv = _qkv_call(x2d, wqkv, ln1_w.reshape(1, D), cos_t, sin_t)
    q3 = qkv[:, :H * DH].reshape(T, H, DH).transpose(1, 0, 2)
    k3 = qkv[:, H * DH:(H + KV) * DH].reshape(T, KV, DH).transpose(1, 0, 2)
    v3 = qkv[:, (H + KV) * DH:].reshape(T, KV, DH).transpose(1, 0, 2)
    o3 = _attn_call(q3, k3, v3)
    o = o3.transpose(1, 0, 2).reshape(T, H * DH)
    h1, xn2, logits, route = _oproj_call(o, Wo, x2d, ln2_w.reshape(1, D), Wg)

    pos2, be64 = _meta_call(route)
    block_expert = be64[0, :NB]
    p0 = pos2[:, 0]
    p1 = pos2[:, 1]

    xs = _sc_dispatch_call(xn2, p0, p1)
    y = _gmm_call(xs, w1, w3, w2, block_expert)

    out2d = _sc_combine_call(h1, y, p0, p1, route[:, 2], route[:, 3])

    out = out2d.reshape(B, S, D)
    new_gate = gate_logits.at[0].set(logits)
    return (out, position_ids, new_gate)
